# Initial kernel scaffold; baseline (speedup 1.0000x reference)
import jax
import jax.numpy as jnp
from jax.experimental import pallas as pl


def kernel(hidden_states, W_router, W1, b1, W2, b2):
    raise NotImplementedError("write your pallas kernel here")


# speedup vs baseline: 1.4004x; 1.4004x over previous
import functools

import jax
import jax.numpy as jnp
from jax import lax
from jax.experimental import pallas as pl
from jax.experimental.pallas import tpu as pltpu

T = 4096
H = 1024
E = 8
F = 2048
K = 2
A = T * K
TM = 256
A_PAD = A + E * TM
NT = A_PAD // TM
CB = 256


def _routing_body(x_ref, wr_ref, pos_ref, gate_ref, emap_ref, e_scr, rank_scr):
    x = x_ref[...]
    logits = jnp.dot(x, wr_ref[...], preferred_element_type=jnp.float32)
    iota_e = lax.broadcasted_iota(jnp.int32, (1, E), 1).astype(jnp.float32)
    m1 = jnp.max(logits, axis=1, keepdims=True)
    i1 = jnp.min(jnp.where(logits == m1, iota_e, float(E)), axis=1, keepdims=True)
    masked = jnp.where(iota_e == i1, -jnp.inf, logits)
    m2 = jnp.max(masked, axis=1, keepdims=True)
    i2 = jnp.min(jnp.where(masked == m2, iota_e, float(E)), axis=1, keepdims=True)
    d = jnp.exp(m2 - m1)
    g1 = 1.0 / (1.0 + d)
    g2 = d / (1.0 + d)

    e_scr[0:T, :] = i1
    e_scr[T:A, :] = i2
    gate_ref[0:T, :] = g1
    gate_ref[T:A, :] = g2

    iota_r = lax.broadcasted_iota(jnp.int32, (CB, CB), 0)
    iota_c = lax.broadcasted_iota(jnp.int32, (CB, CB), 1)
    l_strict = (iota_r > iota_c).astype(jnp.float32)

    def blk(i, carry):
        eb = e_scr[pl.ds(i * CB, CB), :]
        cb = (eb == iota_e).astype(jnp.float32)
        excl = jnp.dot(l_strict, cb, preferred_element_type=jnp.float32)
        rank = jnp.sum((excl + carry) * cb, axis=1, keepdims=True)
        rank_scr[pl.ds(i * CB, CB), :] = rank
        return carry + jnp.sum(cb, axis=0, keepdims=True)

    counts = lax.fori_loop(0, A // CB, blk, jnp.zeros((1, E), jnp.float32))

    counts_i = counts.astype(jnp.int32)
    cap = ((counts_i + (TM - 1)) >> 8) << 8
    s = cap
    for sh in (1, 2, 4):
        s = s + jnp.concatenate([jnp.zeros((1, sh), jnp.int32), s[:, : E - sh]], axis=1)
    off_pad = (s - cap).astype(jnp.float32)
    ends = s

    e_all = e_scr[...]
    c_all = (e_all == iota_e).astype(jnp.float32)
    off_a = jnp.sum(c_all * off_pad, axis=1, keepdims=True)
    pos_ref[...] = (off_a + rank_scr[...]).astype(jnp.int32)

    tile_start = lax.broadcasted_iota(jnp.int32, (NT, 1), 0) * TM
    e_of_tile = jnp.sum((tile_start >= ends).astype(jnp.int32), axis=1, keepdims=True)
    emap_ref[...] = jnp.minimum(e_of_tile, E - 1)


def _routing(x, w_router, interpret=False):
    return pl.pallas_call(
        _routing_body,
        out_shape=(
            jax.ShapeDtypeStruct((A, 1), jnp.int32),
            jax.ShapeDtypeStruct((A, 1), jnp.float32),
            jax.ShapeDtypeStruct((NT, 1), jnp.int32),
        ),
        scratch_shapes=[
            pltpu.VMEM((A, 1), jnp.float32),
            pltpu.VMEM((A, 1), jnp.float32),
        ],
        interpret=interpret,
    )(x, w_router)


def _ffn_body(emap_ref, x_ref, g_ref, w1_ref, b1_ref, w2_ref, b2_ref, out_ref):
    xt = x_ref[...]
    h = jnp.dot(xt, w1_ref[0], preferred_element_type=jnp.float32) + b1_ref[0]
    h = jnp.maximum(h, 0.0)
    y = jnp.dot(h, w2_ref[0], preferred_element_type=jnp.float32) + b2_ref[0]
    out_ref[...] = y * g_ref[...]


def _ffn(x_sorted, gate_sorted, emap, w1, b1, w2, b2, interpret=False):
    grid_spec = pltpu.PrefetchScalarGridSpec(
        num_scalar_prefetch=1,
        grid=(NT,),
        in_specs=[
            pl.BlockSpec((TM, H), lambda m, emap: (m, 0)),
            pl.BlockSpec((TM, 1), lambda m, emap: (m, 0)),
            pl.BlockSpec((1, H, F), lambda m, emap: (emap[m], 0, 0)),
            pl.BlockSpec((1, 1, F), lambda m, emap: (emap[m], 0, 0)),
            pl.BlockSpec((1, F, H), lambda m, emap: (emap[m], 0, 0)),
            pl.BlockSpec((1, 1, H), lambda m, emap: (emap[m], 0, 0)),
        ],
        out_specs=pl.BlockSpec((TM, H), lambda m, emap: (m, 0)),
    )
    return pl.pallas_call(
        _ffn_body,
        grid_spec=grid_spec,
        out_shape=jax.ShapeDtypeStruct((A_PAD, H), jnp.float32),
        interpret=interpret,
    )(emap, x_sorted, gate_sorted, w1, b1.reshape(E, 1, F), w2, b2.reshape(E, 1, H))


def _build_sorted(pos, gates):
    tok = jnp.concatenate([jnp.arange(T, dtype=jnp.int32)] * K)
    idx_sorted = jnp.zeros((A_PAD,), jnp.int32).at[pos].set(tok)
    gate_sorted = jnp.zeros((A_PAD,), jnp.float32).at[pos].set(gates)
    return idx_sorted, gate_sorted


def _gather_rows(x, idx_sorted):
    return x[idx_sorted]


def _combine(y_sorted, pos):
    return y_sorted[pos[:T]] + y_sorted[pos[T:]]


@jax.jit
def kernel(hidden_states, W_router, W1, b1, W2, b2):
    Bsz, Seq, Hdim = hidden_states.shape
    x = hidden_states.reshape(-1, Hdim)
    pos2, gates2, emap2 = _routing(x, W_router)
    pos = pos2.reshape(A)
    gates = gates2.reshape(A)
    emap = emap2.reshape(NT)
    idx_sorted, gate_sorted = _build_sorted(pos, gates)
    x_sorted = _gather_rows(x, idx_sorted)
    y_sorted = _ffn(x_sorted, gate_sorted.reshape(A_PAD, 1), emap, W1, b1, W2, b2)
    y = _combine(y_sorted, pos)
    return y.reshape(Bsz, Seq, Hdim)
